# baseline (device time: 23325 ns/iter reference)
import jax
import jax.numpy as jnp
from jax import lax
from jax.experimental import pallas as pl
from jax.experimental.pallas import tpu as pltpu

Z = 4


def kernel(x):
    m, n = x.shape
    blk = n // Z
    qm = m // 4
    hqm = qm // 2

    def body(x_ref, out_ref, xb_ref, xq_ref, xo_ref, local_sems,
             zs_sems, zr_sems, ds_sems, dr_sems, fs_sems, fr_sems):
        my_x = lax.axis_index("x")
        my_y = lax.axis_index("y")
        my_z = lax.axis_index("z")
        q = 2 * my_y + my_x
        qx = 2 * my_y + 1 - my_x
        qy = 2 * (1 - my_y) + my_x
        x_nbr = (1 - my_x, my_y, my_z)
        y_nbr = (my_x, 1 - my_y, my_z)

        qcopies = []
        for d in range(1, Z):
            tz = (my_z + d) % Z
            c = pltpu.make_async_copy(
                x_ref.at[pl.ds(q * qm, qm), pl.ds(tz * blk, blk)],
                xq_ref.at[d - 1],
                local_sems.at[d - 1],
            )
            c.start()
            qcopies.append(c)
        oc = pltpu.make_async_copy(
            x_ref.at[:, pl.ds(my_z * blk, blk)], xo_ref, local_sems.at[3]
        )
        oc.start()
        for d in range(1, Z):
            qcopies[d - 1].wait()
            xb_ref[d - 1] = xq_ref[d - 1].astype(jnp.bfloat16)

        barrier_sem = pltpu.get_barrier_semaphore()
        for d in range(1, Z):
            pl.semaphore_signal(
                barrier_sem, inc=1,
                device_id=(my_x, my_y, (my_z + d) % Z),
                device_id_type=pl.DeviceIdType.MESH,
            )
        for nbr in (x_nbr, y_nbr):
            pl.semaphore_signal(
                barrier_sem, inc=1,
                device_id=nbr, device_id_type=pl.DeviceIdType.MESH,
            )
        pl.semaphore_wait(barrier_sem, 5)

        z_rdmas = []
        for d in range(1, Z):
            tz = (my_z + d) % Z
            r = pltpu.make_async_remote_copy(
                src_ref=xb_ref.at[d - 1],
                dst_ref=out_ref.at[pl.ds(my_z * m + q * qm, qm), :],
                send_sem=zs_sems.at[d - 1],
                recv_sem=zr_sems.at[d - 1],
                device_id=(my_x, my_y, tz),
                device_id_type=pl.DeviceIdType.MESH,
            )
            r.start()
            z_rdmas.append(r)

        oc.wait()
        out_ref[pl.ds(my_z * m, m), :] = xo_ref[...].astype(jnp.bfloat16)

        def fan_out(d):
            sz = (my_z - d) % Z
            z_rdmas[d - 1].wait_recv()
            rows = pl.ds(sz * m + q * qm, qm)
            for i, nbr in enumerate((x_nbr, y_nbr)):
                r = pltpu.make_async_remote_copy(
                    src_ref=out_ref.at[rows, :],
                    dst_ref=out_ref.at[rows, :],
                    send_sem=ds_sems.at[(d - 1) * 2 + i],
                    recv_sem=dr_sems.at[(d - 1) * 2 + i],
                    device_id=nbr,
                    device_id_type=pl.DeviceIdType.MESH,
                )
                r.start()

        def forward(d):
            sz = (my_z - d) % Z
            for i, (src_q, off, nbr) in enumerate(
                ((qx, 0, y_nbr), (qy, hqm, x_nbr))
            ):
                recv = pltpu.make_async_remote_copy(
                    src_ref=out_ref.at[pl.ds(0, qm), :],
                    dst_ref=out_ref.at[pl.ds(sz * m + src_q * qm, qm), :],
                    send_sem=ds_sems.at[(d - 1) * 2 + i],
                    recv_sem=dr_sems.at[(d - 1) * 2 + i],
                    device_id=nbr,
                    device_id_type=pl.DeviceIdType.MESH,
                )
                recv.wait_recv()
                rows = pl.ds(sz * m + src_q * qm + off, hqm)
                r = pltpu.make_async_remote_copy(
                    src_ref=out_ref.at[rows, :],
                    dst_ref=out_ref.at[rows, :],
                    send_sem=fs_sems.at[(d - 1) * 2 + i],
                    recv_sem=fr_sems.at[(d - 1) * 2 + i],
                    device_id=nbr,
                    device_id_type=pl.DeviceIdType.MESH,
                )
                r.start()

        orders = {0: (3, 2, 1), 1: (1, 3, 2), 2: (1, 3, 2), 3: (1, 2, 3)}
        for zi, (a, b, c) in orders.items():
            @pl.when(my_z == zi)
            def _(a=a, b=b, c=c):
                fan_out(a)
                fan_out(b)
                forward(a)
                fan_out(c)
                forward(b)
                forward(c)

        for d in range(1, Z):
            for i, nbr in enumerate((y_nbr, x_nbr)):
                r = pltpu.make_async_remote_copy(
                    src_ref=out_ref.at[pl.ds(0, hqm), :],
                    dst_ref=out_ref.at[pl.ds(0, hqm), :],
                    send_sem=fs_sems.at[(d - 1) * 2 + i],
                    recv_sem=fr_sems.at[(d - 1) * 2 + i],
                    device_id=nbr,
                    device_id_type=pl.DeviceIdType.MESH,
                )
                r.wait_recv()
        for r in z_rdmas:
            r.wait_send()
        for d in range(1, Z):
            for i in range(2):
                sendwait = pltpu.make_async_remote_copy(
                    src_ref=out_ref.at[pl.ds(0, qm), :],
                    dst_ref=out_ref.at[pl.ds(0, qm), :],
                    send_sem=ds_sems.at[(d - 1) * 2 + i],
                    recv_sem=dr_sems.at[(d - 1) * 2 + i],
                    device_id=x_nbr,
                    device_id_type=pl.DeviceIdType.MESH,
                )
                sendwait.wait_send()
                fwait = pltpu.make_async_remote_copy(
                    src_ref=out_ref.at[pl.ds(0, hqm), :],
                    dst_ref=out_ref.at[pl.ds(0, hqm), :],
                    send_sem=fs_sems.at[(d - 1) * 2 + i],
                    recv_sem=fr_sems.at[(d - 1) * 2 + i],
                    device_id=x_nbr,
                    device_id_type=pl.DeviceIdType.MESH,
                )
                fwait.wait_send()

    out_shape = jax.ShapeDtypeStruct((Z * m, blk), jnp.bfloat16)
    return pl.pallas_call(
        body,
        out_shape=out_shape,
        in_specs=[pl.BlockSpec(memory_space=pl.ANY)],
        out_specs=pl.BlockSpec(memory_space=pltpu.VMEM),
        scratch_shapes=[
            pltpu.VMEM((Z - 1, qm, blk), jnp.bfloat16),
            pltpu.VMEM((Z - 1, qm, blk), jnp.float32),
            pltpu.VMEM((m, blk), jnp.float32),
            pltpu.SemaphoreType.DMA((Z,)),
            pltpu.SemaphoreType.DMA((Z - 1,)),
            pltpu.SemaphoreType.DMA((Z - 1,)),
            pltpu.SemaphoreType.DMA((2 * (Z - 1),)),
            pltpu.SemaphoreType.DMA((2 * (Z - 1),)),
            pltpu.SemaphoreType.DMA((2 * (Z - 1),)),
            pltpu.SemaphoreType.DMA((2 * (Z - 1),)),
        ],
        compiler_params=pltpu.CompilerParams(collective_id=0),
    )(x)


# device time: 22850 ns/iter; 1.0208x vs baseline; 1.0208x over previous
import jax
import jax.numpy as jnp
from jax import lax
from jax.experimental import pallas as pl
from jax.experimental.pallas import tpu as pltpu

Z = 4


def kernel(x):
    m, n = x.shape
    blk = n // Z
    qm = m // 4

    def body(x_ref, out_ref, xb_ref, zs_sems, zr_sems, ps_sems, pr_sems):
        my_x = lax.axis_index("x")
        my_y = lax.axis_index("y")
        my_z = lax.axis_index("z")
        q = 2 * my_y + my_x

        xb_ref[...] = x_ref[...].astype(jnp.bfloat16)

        peers = (
            (1 - my_x, my_y, my_z),
            (my_x, 1 - my_y, my_z),
            (1 - my_x, 1 - my_y, my_z),
        )

        barrier_sem = pltpu.get_barrier_semaphore()
        for d in range(1, Z):
            pl.semaphore_signal(
                barrier_sem, inc=1,
                device_id=(my_x, my_y, (my_z + d) % Z),
                device_id_type=pl.DeviceIdType.MESH,
            )
        for nbr in peers:
            pl.semaphore_signal(
                barrier_sem, inc=1,
                device_id=nbr, device_id_type=pl.DeviceIdType.MESH,
            )
        pl.semaphore_wait(barrier_sem, 6)

        z_rdmas = []
        for d in range(1, Z):
            tz = (my_z + d) % Z
            r = pltpu.make_async_remote_copy(
                src_ref=xb_ref.at[pl.ds(q * qm, qm), pl.ds(tz * blk, blk)],
                dst_ref=out_ref.at[pl.ds(my_z * m + q * qm, qm), :],
                send_sem=zs_sems.at[d - 1],
                recv_sem=zr_sems.at[d - 1],
                device_id=(my_x, my_y, tz),
                device_id_type=pl.DeviceIdType.MESH,
            )
            r.start()
            z_rdmas.append(r)

        out_ref[pl.ds(my_z * m, m), :] = xb_ref[:, pl.ds(my_z * blk, blk)]

        p_rdmas = []
        for d in range(1, Z):
            sz = (my_z - d) % Z
            z_rdmas[d - 1].wait_recv()
            rows = pl.ds(sz * m + q * qm, qm)
            for i, nbr in enumerate(peers):
                slot = (d - 1) * 3 + i
                r = pltpu.make_async_remote_copy(
                    src_ref=out_ref.at[rows, :],
                    dst_ref=out_ref.at[rows, :],
                    send_sem=ps_sems.at[slot],
                    recv_sem=pr_sems.at[slot],
                    device_id=nbr,
                    device_id_type=pl.DeviceIdType.MESH,
                )
                r.start()
                p_rdmas.append(r)

        for r in p_rdmas:
            r.wait_recv()
        for r in z_rdmas + p_rdmas:
            r.wait_send()

    out_shape = jax.ShapeDtypeStruct((Z * m, blk), jnp.bfloat16)
    return pl.pallas_call(
        body,
        out_shape=out_shape,
        in_specs=[pl.BlockSpec(memory_space=pltpu.VMEM)],
        out_specs=pl.BlockSpec(memory_space=pltpu.VMEM),
        scratch_shapes=[
            pltpu.VMEM((m, n), jnp.bfloat16),
            pltpu.SemaphoreType.DMA((Z - 1,)),
            pltpu.SemaphoreType.DMA((Z - 1,)),
            pltpu.SemaphoreType.DMA((3 * (Z - 1),)),
            pltpu.SemaphoreType.DMA((3 * (Z - 1),)),
        ],
        compiler_params=pltpu.CompilerParams(collective_id=0),
    )(x)


# device time: 22617 ns/iter; 1.0313x vs baseline; 1.0103x over previous
import jax
import jax.numpy as jnp
from jax import lax
from jax.experimental import pallas as pl
from jax.experimental.pallas import tpu as pltpu

Z = 4


def kernel(x):
    m, n = x.shape
    blk = n // Z
    qm = m // 4

    def body(x_ref, out_ref, xb_ref, zs_sems, zr_sems, ps_sems, pr_sems):
        my_x = lax.axis_index("x")
        my_y = lax.axis_index("y")
        my_z = lax.axis_index("z")
        q = 2 * my_y + my_x

        for d in range(1, Z):
            tz = (my_z + d) % Z
            xb_ref[d - 1] = x_ref[
                pl.ds(q * qm, qm), pl.ds(tz * blk, blk)
            ].astype(jnp.bfloat16)

        peers = (
            (1 - my_x, my_y, my_z),
            (my_x, 1 - my_y, my_z),
            (1 - my_x, 1 - my_y, my_z),
        )

        barrier_sem = pltpu.get_barrier_semaphore()
        for d in range(1, Z):
            pl.semaphore_signal(
                barrier_sem, inc=1,
                device_id=(my_x, my_y, (my_z + d) % Z),
                device_id_type=pl.DeviceIdType.MESH,
            )
        for nbr in peers:
            pl.semaphore_signal(
                barrier_sem, inc=1,
                device_id=nbr, device_id_type=pl.DeviceIdType.MESH,
            )
        pl.semaphore_wait(barrier_sem, 6)

        z_rdmas = []
        for d in range(1, Z):
            tz = (my_z + d) % Z
            r = pltpu.make_async_remote_copy(
                src_ref=xb_ref.at[d - 1],
                dst_ref=out_ref.at[pl.ds(my_z * m + q * qm, qm), :],
                send_sem=zs_sems.at[d - 1],
                recv_sem=zr_sems.at[d - 1],
                device_id=(my_x, my_y, tz),
                device_id_type=pl.DeviceIdType.MESH,
            )
            r.start()
            z_rdmas.append(r)

        out_ref[pl.ds(my_z * m, m), :] = (
            x_ref[:, pl.ds(my_z * blk, blk)].astype(jnp.bfloat16)
        )

        p_rdmas = []
        for d in range(1, Z):
            sz = (my_z - d) % Z
            z_rdmas[d - 1].wait_recv()
            rows = pl.ds(sz * m + q * qm, qm)
            for i, nbr in enumerate(peers):
                slot = (d - 1) * 3 + i
                r = pltpu.make_async_remote_copy(
                    src_ref=out_ref.at[rows, :],
                    dst_ref=out_ref.at[rows, :],
                    send_sem=ps_sems.at[slot],
                    recv_sem=pr_sems.at[slot],
                    device_id=nbr,
                    device_id_type=pl.DeviceIdType.MESH,
                )
                r.start()
                p_rdmas.append(r)

        for r in p_rdmas:
            r.wait_recv()
        for r in z_rdmas + p_rdmas:
            r.wait_send()

    out_shape = jax.ShapeDtypeStruct((Z * m, blk), jnp.bfloat16)
    return pl.pallas_call(
        body,
        out_shape=out_shape,
        in_specs=[pl.BlockSpec(memory_space=pltpu.VMEM)],
        out_specs=pl.BlockSpec(memory_space=pltpu.VMEM),
        scratch_shapes=[
            pltpu.VMEM((Z - 1, qm, blk), jnp.bfloat16),
            pltpu.SemaphoreType.DMA((Z - 1,)),
            pltpu.SemaphoreType.DMA((Z - 1,)),
            pltpu.SemaphoreType.DMA((3 * (Z - 1),)),
            pltpu.SemaphoreType.DMA((3 * (Z - 1),)),
        ],
        compiler_params=pltpu.CompilerParams(collective_id=0),
    )(x)


# device time: 22133 ns/iter; 1.0539x vs baseline; 1.0219x over previous
import jax
import jax.numpy as jnp
from jax import lax
from jax.experimental import pallas as pl
from jax.experimental.pallas import tpu as pltpu

Z = 4
H = 2


def kernel(x):
    m, n = x.shape
    blk = n // Z
    qm = m // 4
    hm = qm // H

    def body(x_ref, out_ref, xb_ref, zs_sems, zr_sems, ps_sems, pr_sems):
        my_x = lax.axis_index("x")
        my_y = lax.axis_index("y")
        my_z = lax.axis_index("z")
        q = 2 * my_y + my_x

        for d in range(1, Z):
            tz = (my_z + d) % Z
            for h in range(H):
                xb_ref[(d - 1) * H + h] = x_ref[
                    pl.ds(q * qm + h * hm, hm), pl.ds(tz * blk, blk)
                ].astype(jnp.bfloat16)

        peers = (
            (1 - my_x, 1 - my_y, my_z),
            (1 - my_x, my_y, my_z),
            (my_x, 1 - my_y, my_z),
        )

        barrier_sem = pltpu.get_barrier_semaphore()
        for d in range(1, Z):
            pl.semaphore_signal(
                barrier_sem, inc=1,
                device_id=(my_x, my_y, (my_z + d) % Z),
                device_id_type=pl.DeviceIdType.MESH,
            )
        for nbr in peers:
            pl.semaphore_signal(
                barrier_sem, inc=1,
                device_id=nbr, device_id_type=pl.DeviceIdType.MESH,
            )
        pl.semaphore_wait(barrier_sem, 6)

        z_rdmas = []
        for d in range(1, Z):
            tz = (my_z + d) % Z
            for h in range(H):
                slot = (d - 1) * H + h
                r = pltpu.make_async_remote_copy(
                    src_ref=xb_ref.at[slot],
                    dst_ref=out_ref.at[
                        pl.ds(my_z * m + q * qm + h * hm, hm), :
                    ],
                    send_sem=zs_sems.at[slot],
                    recv_sem=zr_sems.at[slot],
                    device_id=(my_x, my_y, tz),
                    device_id_type=pl.DeviceIdType.MESH,
                )
                r.start()
                z_rdmas.append(r)

        out_ref[pl.ds(my_z * m, m), :] = (
            x_ref[:, pl.ds(my_z * blk, blk)].astype(jnp.bfloat16)
        )

        p_rdmas = []
        for d in range(1, Z):
            sz = (my_z - d) % Z
            for h in range(H):
                zslot = (d - 1) * H + h
                z_rdmas[zslot].wait_recv()
                rows = pl.ds(sz * m + q * qm + h * hm, hm)
                for i, nbr in enumerate(peers):
                    slot = zslot * 3 + i
                    r = pltpu.make_async_remote_copy(
                        src_ref=out_ref.at[rows, :],
                        dst_ref=out_ref.at[rows, :],
                        send_sem=ps_sems.at[slot],
                        recv_sem=pr_sems.at[slot],
                        device_id=nbr,
                        device_id_type=pl.DeviceIdType.MESH,
                    )
                    r.start()
                    p_rdmas.append(r)

        for r in p_rdmas:
            r.wait_recv()
        for r in z_rdmas + p_rdmas:
            r.wait_send()

    out_shape = jax.ShapeDtypeStruct((Z * m, blk), jnp.bfloat16)
    return pl.pallas_call(
        body,
        out_shape=out_shape,
        in_specs=[pl.BlockSpec(memory_space=pltpu.VMEM)],
        out_specs=pl.BlockSpec(memory_space=pltpu.VMEM),
        scratch_shapes=[
            pltpu.VMEM((H * (Z - 1), hm, blk), jnp.bfloat16),
            pltpu.SemaphoreType.DMA((H * (Z - 1),)),
            pltpu.SemaphoreType.DMA((H * (Z - 1),)),
            pltpu.SemaphoreType.DMA((3 * H * (Z - 1),)),
            pltpu.SemaphoreType.DMA((3 * H * (Z - 1),)),
        ],
        compiler_params=pltpu.CompilerParams(collective_id=0),
    )(x)
